# Initial kernel scaffold; baseline (speedup 1.0000x reference)
import jax
import jax.numpy as jnp
from jax.experimental import pallas as pl


def kernel(x, E, num_batch, batch_seg, Wq, bq, Wk, Wv, W1, W2, Wl):
    raise NotImplementedError("write your pallas kernel here")


# speedup vs baseline: 6.7158x; 6.7158x over previous
import numpy as np
import jax
import jax.numpy as jnp
from jax.experimental import pallas as pl
from jax.experimental.pallas import tpu as pltpu

F = 128
BLK = 2000


def _softplus(w):
    return jnp.maximum(w, 0.0) + jnp.log(1.0 + jnp.exp(-jnp.abs(w)))


def _swish(u):
    return u * (1.0 / (1.0 + jnp.exp(-u)))


def _dot(a, b, dims):
    return jax.lax.dot_general(a, b, (dims, ((), ())),
                               preferred_element_type=jnp.float32)


def _body_a(seg_ref, x_ref, esc_ref, wqk_ref, bqk_ref, anorm_ref, a_ref, *, nseg):
    i = pl.program_id(0)
    x = x_ref[...]
    seg = seg_ref[0]
    t = _dot(wqk_ref[...], x, ((1,), (1,)))
    t = t + bqk_ref[0]
    ids = jax.lax.broadcasted_iota(jnp.int32, (nseg, BLK), 0)
    oh = (ids == seg).astype(jnp.float32)
    esc_g = _dot(esc_ref[...], oh, ((1,), (0,)))
    a = _softplus(esc_g * t)
    a_ref[...] = a.reshape(1, 1, BLK)
    pan = _dot(oh, a, ((1,), (1,)))

    @pl.when(i == 0)
    def _():
        anorm_ref[...] = pan

    @pl.when(i > 0)
    def _():
        anorm_ref[...] += pan


def _body_c(seg_ref, a_ref, anorm_ref, etab_ref, wv_ref, w1_ref, w2_ref,
            wl_ref, out_ref, *, nseg):
    seg = seg_ref[0]
    a = a_ref[0]
    ids = jax.lax.broadcasted_iota(jnp.int32, (nseg, BLK), 0)
    oh = (ids == seg).astype(jnp.float32)
    anorm_g = _dot(anorm_ref[...], oh, ((0,), (0,)))
    e_g = _dot(etab_ref[...], oh, ((0,), (0,)))
    coeff = a / (anorm_g + 1e-8) * e_g
    scaled = _dot(coeff, wv_ref[...], ((0,), (0,)))
    s1 = _swish(scaled)
    u = _dot(s1, w1_ref[...], ((1,), (1,)))
    s2 = _swish(u)
    h = scaled + _dot(s2, w2_ref[...], ((1,), (1,)))
    out_ref[...] = _dot(_swish(h), wl_ref[...], ((1,), (1,)))


def kernel(x, E, num_batch, batch_seg, Wq, bq, Wk, Wv, W1, W2, Wl):
    import functools
    N = x.shape[0]
    nseg = E.shape[0]
    nblk = N // BLK
    assert nblk * BLK == N
    inv = np.float32(1.0 / np.sqrt(F))

    wqk = (Wq.T @ Wk).reshape(1, F) * inv
    bqk = (bq @ Wk).reshape(1, 1) * inv
    e = jnp.abs(E)
    esc = (e / jnp.maximum(e, 1.0)).reshape(1, nseg)
    etab = e.reshape(nseg, 1)
    wv = Wv.reshape(1, F)
    seg3 = batch_seg.reshape(nblk, 1, BLK)

    anorm, a3 = pl.pallas_call(
        functools.partial(_body_a, nseg=nseg),
        grid=(nblk,),
        in_specs=[
            pl.BlockSpec((1, 1, BLK), lambda i: (i, 0, 0)),
            pl.BlockSpec((BLK, F), lambda i: (i, 0)),
            pl.BlockSpec((1, nseg), lambda i: (0, 0)),
            pl.BlockSpec((1, F), lambda i: (0, 0)),
            pl.BlockSpec((1, 1), lambda i: (0, 0)),
        ],
        out_specs=[
            pl.BlockSpec((nseg, 1), lambda i: (0, 0)),
            pl.BlockSpec((1, 1, BLK), lambda i: (i, 0, 0)),
        ],
        out_shape=[
            jax.ShapeDtypeStruct((nseg, 1), jnp.float32),
            jax.ShapeDtypeStruct((nblk, 1, BLK), jnp.float32),
        ],
    )(seg3, x, esc, wqk, bqk)

    out = pl.pallas_call(
        functools.partial(_body_c, nseg=nseg),
        grid=(nblk,),
        in_specs=[
            pl.BlockSpec((1, 1, BLK), lambda i: (i, 0, 0)),
            pl.BlockSpec((1, 1, BLK), lambda i: (i, 0, 0)),
            pl.BlockSpec((nseg, 1), lambda i: (0, 0)),
            pl.BlockSpec((nseg, 1), lambda i: (0, 0)),
            pl.BlockSpec((1, F), lambda i: (0, 0)),
            pl.BlockSpec((F, F), lambda i: (0, 0)),
            pl.BlockSpec((F, F), lambda i: (0, 0)),
            pl.BlockSpec((F, F), lambda i: (0, 0)),
        ],
        out_specs=pl.BlockSpec((BLK, F), lambda i: (i, 0)),
        out_shape=jax.ShapeDtypeStruct((N, F), jnp.float32),
    )(seg3, a3, anorm, etab, wv, W1, W2, Wl)
    return out
